# baseline (speedup 1.0000x reference)
import jax
import jax.numpy as jnp
from jax.experimental import pallas as pl
from jax.experimental.pallas import tpu as pltpu

B = 512
N = 77
IN_FEATS = 256
HIDDEN = 512
OUT_FEATS = 256
E = 2464

BB = 32


def _build_ahat_kernel(ei_ref, ahat_ref):
    src = ei_ref[0:1, :]
    dst = ei_ref[1:2, :]
    rows = jax.lax.broadcasted_iota(jnp.int32, (N, E), 0)
    onehot_dst = (rows == dst).astype(jnp.float32)
    onehot_src = (rows == src).astype(jnp.float32)
    a = jax.lax.dot_general(
        onehot_dst, onehot_src,
        (((1,), (1,)), ((), ())),
        preferred_element_type=jnp.float32,
    )
    deg_in = jnp.sum(a, axis=1, keepdims=True)
    deg_out = jnp.sum(a, axis=0, keepdims=True)
    norm_dst = jax.lax.rsqrt(jnp.maximum(deg_in, 1.0))
    norm_src = jax.lax.rsqrt(jnp.maximum(deg_out, 1.0))
    ahat_ref[:, :] = (a * norm_dst * norm_src).astype(jnp.bfloat16)


def _mm(a, b):
    return jax.lax.dot_general(
        a, b, (((1,), (0,)), ((), ())), preferred_element_type=jnp.float32
    )


def _bmm_ahat(ahat_b, v):
    return jax.lax.dot_general(
        ahat_b, v, (((2,), (1,)), ((0,), (0,))),
        preferred_element_type=jnp.float32,
    )


def _gcn_kernel(x_ref, ahat_ref, w1_ref, b1_ref, w2_ref, b2_ref, out_ref):
    ahat_b = jnp.broadcast_to(ahat_ref[:, :][None], (BB, N, N))
    x = x_ref[...]
    y = _bmm_ahat(ahat_b, x).reshape(BB * N, IN_FEATS)
    h = jnp.maximum(_mm(y.astype(jnp.bfloat16), w1_ref[:, :]) + b1_ref[:, :],
                    0.0)
    t = _mm(h.astype(jnp.bfloat16), w2_ref[:, :]).reshape(BB, N, OUT_FEATS)
    out_ref[...] = _bmm_ahat(ahat_b, t.astype(jnp.bfloat16)) + b2_ref[:, :][None]


def kernel(in_feat, edge_index, W1, b1, W2, b2):
    ahat = pl.pallas_call(
        _build_ahat_kernel,
        out_shape=jax.ShapeDtypeStruct((N, N), jnp.bfloat16),
    )(edge_index)

    grid = (B // BB,)
    out = pl.pallas_call(
        _gcn_kernel,
        grid=grid,
        in_specs=[
            pl.BlockSpec((BB, N, IN_FEATS), lambda i: (i, 0, 0)),
            pl.BlockSpec((N, N), lambda i: (0, 0)),
            pl.BlockSpec((IN_FEATS, HIDDEN), lambda i: (0, 0)),
            pl.BlockSpec((1, HIDDEN), lambda i: (0, 0)),
            pl.BlockSpec((HIDDEN, OUT_FEATS), lambda i: (0, 0)),
            pl.BlockSpec((1, OUT_FEATS), lambda i: (0, 0)),
        ],
        out_specs=pl.BlockSpec((BB, N, OUT_FEATS), lambda i: (i, 0, 0)),
        out_shape=jax.ShapeDtypeStruct((B, N, OUT_FEATS), jnp.float32),
        compiler_params=pltpu.CompilerParams(
            dimension_semantics=("parallel",),
        ),
    )(in_feat.astype(jnp.bfloat16), ahat, W1.astype(jnp.bfloat16),
      b1.reshape(1, HIDDEN), W2.astype(jnp.bfloat16), b2.reshape(1, OUT_FEATS))
    return out


# speedup vs baseline: 1.1208x; 1.1208x over previous
import jax
import jax.numpy as jnp
from jax.experimental import pallas as pl
from jax.experimental.pallas import tpu as pltpu

B = 512
N = 77
IN_FEATS = 256
HIDDEN = 512
OUT_FEATS = 256
E = 2464

BB = 32


def _build_ahat_kernel(ei_ref, ahat_ref):
    src = ei_ref[0:1, :]
    dst = ei_ref[1:2, :]
    rows = jax.lax.broadcasted_iota(jnp.int32, (N, E), 0)
    onehot_dst = (rows == dst).astype(jnp.float32)
    onehot_src = (rows == src).astype(jnp.float32)
    a = jax.lax.dot_general(
        onehot_dst, onehot_src,
        (((1,), (1,)), ((), ())),
        preferred_element_type=jnp.float32,
    )
    deg_in = jnp.sum(a, axis=1, keepdims=True)
    deg_out = jnp.sum(a, axis=0, keepdims=True)
    norm_dst = jax.lax.rsqrt(jnp.maximum(deg_in, 1.0))
    norm_src = jax.lax.rsqrt(jnp.maximum(deg_out, 1.0))
    ahat_ref[:, :] = (a * norm_dst * norm_src).astype(jnp.bfloat16)


def _mm(a, b):
    return jax.lax.dot_general(
        a, b, (((1,), (0,)), ((), ())), preferred_element_type=jnp.float32
    )


def _bmm_ahat(ahat_b, v):
    return jax.lax.dot_general(
        ahat_b, v, (((2,), (1,)), ((0,), (0,))),
        preferred_element_type=jnp.float32,
    )


def _gcn_kernel(x_ref, ahat_ref, w1_ref, b1_ref, w2_ref, b2_ref, out_ref):
    ahat_b = jnp.broadcast_to(ahat_ref[:, :][None], (BB, N, N))
    x = x_ref[...].astype(jnp.bfloat16)
    y = _bmm_ahat(ahat_b, x).reshape(BB * N, IN_FEATS)
    h = jnp.maximum(_mm(y.astype(jnp.bfloat16), w1_ref[:, :]) + b1_ref[:, :],
                    0.0)
    t = _mm(h.astype(jnp.bfloat16), w2_ref[:, :]).reshape(BB, N, OUT_FEATS)
    out_ref[...] = _bmm_ahat(ahat_b, t.astype(jnp.bfloat16)) + b2_ref[:, :][None]


def kernel(in_feat, edge_index, W1, b1, W2, b2):
    ahat = pl.pallas_call(
        _build_ahat_kernel,
        out_shape=jax.ShapeDtypeStruct((N, N), jnp.bfloat16),
    )(edge_index)

    grid = (B // BB,)
    out = pl.pallas_call(
        _gcn_kernel,
        grid=grid,
        in_specs=[
            pl.BlockSpec((BB, N, IN_FEATS), lambda i: (i, 0, 0)),
            pl.BlockSpec((N, N), lambda i: (0, 0)),
            pl.BlockSpec((IN_FEATS, HIDDEN), lambda i: (0, 0)),
            pl.BlockSpec((1, HIDDEN), lambda i: (0, 0)),
            pl.BlockSpec((HIDDEN, OUT_FEATS), lambda i: (0, 0)),
            pl.BlockSpec((1, OUT_FEATS), lambda i: (0, 0)),
        ],
        out_specs=pl.BlockSpec((BB, N, OUT_FEATS), lambda i: (i, 0, 0)),
        out_shape=jax.ShapeDtypeStruct((B, N, OUT_FEATS), jnp.float32),
        compiler_params=pltpu.CompilerParams(
            dimension_semantics=("parallel",),
        ),
    )(in_feat, ahat, W1.astype(jnp.bfloat16),
      b1.reshape(1, HIDDEN), W2.astype(jnp.bfloat16), b2.reshape(1, OUT_FEATS))
    return out


# speedup vs baseline: 1.3620x; 1.2152x over previous
import jax
import jax.numpy as jnp
from jax.experimental import pallas as pl
from jax.experimental.pallas import tpu as pltpu

B = 512
N = 77
NP = 80
IN_FEATS = 256
HIDDEN = 512
OUT_FEATS = 256
E = 2464

BB = 32


def _build_ahat_kernel(ei_ref, ahat_ref):
    src = ei_ref[0:1, :]
    dst = ei_ref[1:2, :]
    rows = jax.lax.broadcasted_iota(jnp.int32, (NP, E), 0)
    onehot_dst = (rows == dst).astype(jnp.float32)
    onehot_src = (rows == src).astype(jnp.float32)
    a = jax.lax.dot_general(
        onehot_dst, onehot_src,
        (((1,), (1,)), ((), ())),
        preferred_element_type=jnp.float32,
    )
    deg_in = jnp.sum(a, axis=1, keepdims=True)
    deg_out = jnp.sum(a, axis=0, keepdims=True)
    norm_dst = jax.lax.rsqrt(jnp.maximum(deg_in, 1.0))
    norm_src = jax.lax.rsqrt(jnp.maximum(deg_out, 1.0))
    ahat_ref[:, :] = (a * norm_dst * norm_src).astype(jnp.bfloat16)


def _mm(a, b):
    return jax.lax.dot_general(
        a, b, (((1,), (0,)), ((), ())), preferred_element_type=jnp.float32
    )


def _bmm_ahat(ahat_b, v):
    return jax.lax.dot_general(
        ahat_b, v, (((2,), (1,)), ((0,), (0,))),
        preferred_element_type=jnp.float32,
    )


def _gcn_kernel(x_ref, ahat_ref, w1_ref, b1_ref, w2_ref, b2_ref, out_ref):
    ahat_b = jnp.broadcast_to(ahat_ref[:, :][None], (BB, NP, NP))
    x = x_ref[...].astype(jnp.bfloat16)
    xp = jnp.concatenate(
        [x, jnp.zeros((BB, NP - N, IN_FEATS), jnp.bfloat16)], axis=1)
    y = _bmm_ahat(ahat_b, xp).astype(jnp.bfloat16).reshape(BB * NP, IN_FEATS)
    h = jnp.maximum(_mm(y, w1_ref[:, :]) + b1_ref[:, :],
                    0.0).astype(jnp.bfloat16)
    t = _mm(h, w2_ref[:, :]).astype(jnp.bfloat16).reshape(BB, NP, OUT_FEATS)
    o = _bmm_ahat(ahat_b, t) + b2_ref[:, :][None]
    out_ref[...] = o[:, :N, :]


def kernel(in_feat, edge_index, W1, b1, W2, b2):
    ahat = pl.pallas_call(
        _build_ahat_kernel,
        out_shape=jax.ShapeDtypeStruct((NP, NP), jnp.bfloat16),
    )(edge_index)

    grid = (B // BB,)
    out = pl.pallas_call(
        _gcn_kernel,
        grid=grid,
        in_specs=[
            pl.BlockSpec((BB, N, IN_FEATS), lambda i: (i, 0, 0)),
            pl.BlockSpec((NP, NP), lambda i: (0, 0)),
            pl.BlockSpec((IN_FEATS, HIDDEN), lambda i: (0, 0)),
            pl.BlockSpec((1, HIDDEN), lambda i: (0, 0)),
            pl.BlockSpec((HIDDEN, OUT_FEATS), lambda i: (0, 0)),
            pl.BlockSpec((1, OUT_FEATS), lambda i: (0, 0)),
        ],
        out_specs=pl.BlockSpec((BB, N, OUT_FEATS), lambda i: (i, 0, 0)),
        out_shape=jax.ShapeDtypeStruct((B, N, OUT_FEATS), jnp.float32),
        compiler_params=pltpu.CompilerParams(
            dimension_semantics=("parallel",),
        ),
    )(in_feat, ahat, W1.astype(jnp.bfloat16),
      b1.astype(jnp.bfloat16).reshape(1, HIDDEN), W2.astype(jnp.bfloat16),
      b2.astype(jnp.bfloat16).reshape(1, OUT_FEATS))
    return out


# speedup vs baseline: 1.5140x; 1.1116x over previous
import jax
import jax.numpy as jnp
from jax.experimental import pallas as pl
from jax.experimental.pallas import tpu as pltpu

B = 512
N = 77
NP = 80
IN_FEATS = 256
HIDDEN = 512
OUT_FEATS = 256
E = 2464

BB = 32


def _build_ahat_kernel(ei_ref, ahat_ref):
    src = ei_ref[0:1, :]
    dst = ei_ref[1:2, :]
    rows = jax.lax.broadcasted_iota(jnp.int32, (NP, E), 0)
    onehot_dst = (rows == dst).astype(jnp.float32)
    onehot_src = (rows == src).astype(jnp.float32)
    a = jax.lax.dot_general(
        onehot_dst, onehot_src,
        (((1,), (1,)), ((), ())),
        preferred_element_type=jnp.float32,
    )
    deg_in = jnp.sum(a, axis=1, keepdims=True)
    deg_out = jnp.sum(a, axis=0, keepdims=True)
    norm_dst = jax.lax.rsqrt(jnp.maximum(deg_in, 1.0))
    norm_src = jax.lax.rsqrt(jnp.maximum(deg_out, 1.0))
    ahat_ref[:, :] = (a * norm_dst * norm_src).astype(jnp.bfloat16)


def _mm(a, b):
    return jax.lax.dot_general(
        a, b, (((1,), (0,)), ((), ())), preferred_element_type=jnp.float32
    )


def _bmm_ahat(ahat_b, v):
    return jax.lax.dot_general(
        ahat_b, v, (((2,), (1,)), ((0,), (0,))),
        preferred_element_type=jnp.float32,
    )


def _gcn_kernel(x_ref, ahat_ref, w1_ref, b1_ref, w2_ref, b2_ref, out_ref):
    ahat_b = jnp.broadcast_to(ahat_ref[:, :][None], (BB, NP, NP))
    x = x_ref[...].astype(jnp.bfloat16)
    xp = jnp.concatenate(
        [x, jnp.zeros((BB, NP - N, IN_FEATS), jnp.bfloat16)], axis=1)
    y = _bmm_ahat(ahat_b, xp).astype(jnp.bfloat16).reshape(BB * NP, IN_FEATS)
    h = jnp.maximum(_mm(y, w1_ref[:, :]) + b1_ref[:, :],
                    0.0).astype(jnp.bfloat16)
    t = _mm(h, w2_ref[:, :]).astype(jnp.bfloat16).reshape(BB, NP, OUT_FEATS)
    o = _bmm_ahat(ahat_b, t) + b2_ref[:, :][None]
    out_ref[...] = o[:, :N, :].astype(jnp.bfloat16)


def kernel(in_feat, edge_index, W1, b1, W2, b2):
    ahat = pl.pallas_call(
        _build_ahat_kernel,
        out_shape=jax.ShapeDtypeStruct((NP, NP), jnp.bfloat16),
    )(edge_index)

    grid = (B // BB,)
    out = pl.pallas_call(
        _gcn_kernel,
        grid=grid,
        in_specs=[
            pl.BlockSpec((BB, N, IN_FEATS), lambda i: (i, 0, 0)),
            pl.BlockSpec((NP, NP), lambda i: (0, 0)),
            pl.BlockSpec((IN_FEATS, HIDDEN), lambda i: (0, 0)),
            pl.BlockSpec((1, HIDDEN), lambda i: (0, 0)),
            pl.BlockSpec((HIDDEN, OUT_FEATS), lambda i: (0, 0)),
            pl.BlockSpec((1, OUT_FEATS), lambda i: (0, 0)),
        ],
        out_specs=pl.BlockSpec((BB, N, OUT_FEATS), lambda i: (i, 0, 0)),
        out_shape=jax.ShapeDtypeStruct((B, N, OUT_FEATS), jnp.bfloat16),
        compiler_params=pltpu.CompilerParams(
            dimension_semantics=("parallel",),
        ),
    )(in_feat, ahat, W1.astype(jnp.bfloat16),
      b1.astype(jnp.bfloat16).reshape(1, HIDDEN), W2.astype(jnp.bfloat16),
      b2.astype(jnp.bfloat16).reshape(1, OUT_FEATS))
    return out.astype(jnp.float32)
